# baseline (speedup 1.0000x reference)
import jax
import jax.numpy as jnp
from jax.experimental import pallas as pl
from jax.experimental.pallas import tpu as pltpu

N = 10000
E = 160000
DT = 1.0
PACK = 8

GRID = 10
NODE_BLOCK = N // GRID
EDGE_BLOCK = E // PACK // GRID


def _fused_kernel(x_ref, w1_ref, b1_ref, w2_ref, b2_ref, wdn_ref, bdn_ref,
                  e_ref, wde_ref, bde_ref, on_ref, oe_ref):
    x = x_ref[...]
    h = jnp.dot(x, w1_ref[...], preferred_element_type=jnp.float32) + b1_ref[...]
    h = h * jax.nn.sigmoid(h)
    h = jnp.dot(h, w2_ref[...], preferred_element_type=jnp.float32) + b2_ref[...]
    h = h * jax.nn.sigmoid(h)
    on_ref[...] = jnp.dot(h, wdn_ref[...], preferred_element_type=jnp.float32) + bdn_ref[...]
    oe_ref[...] = jnp.dot(e_ref[...], wde_ref[...], preferred_element_type=jnp.float32) + bde_ref[...]


def kernel(nodes, edges, senders, receivers, globals_, W_enc_n, b_enc_n, W_enc_e, b_enc_e, W1, b1, W2, b2, Wd_n, bd_n, Wd_e, bd_e):
    d_feat = nodes.shape[1]
    latent = W1.shape[1]
    node_out = Wd_n.shape[1]
    d_edge = edges.shape[1]
    edge_out = Wd_e.shape[1]

    C = E // PACK
    edges_p = jnp.concatenate([edges[k * C:(k + 1) * C] for k in range(PACK)], axis=1)
    Wde_bd = jnp.kron(jnp.eye(PACK, dtype=Wd_e.dtype), Wd_e)
    bde_t = jnp.tile(bd_e, PACK).reshape(1, -1)

    whole = lambda *shape: pl.BlockSpec(shape, lambda i: (0,) * len(shape))

    out_nodes, out_edges_p = pl.pallas_call(
        _fused_kernel,
        grid=(GRID,),
        in_specs=[
            pl.BlockSpec((NODE_BLOCK, d_feat), lambda i: (i, 0)),
            whole(d_feat, latent),
            whole(1, latent),
            whole(latent, latent),
            whole(1, latent),
            whole(latent, node_out),
            whole(1, node_out),
            pl.BlockSpec((EDGE_BLOCK, PACK * d_edge), lambda i: (i, 0)),
            whole(PACK * d_edge, PACK * edge_out),
            whole(1, PACK * edge_out),
        ],
        out_specs=[
            pl.BlockSpec((NODE_BLOCK, node_out), lambda i: (i, 0)),
            pl.BlockSpec((EDGE_BLOCK, PACK * edge_out), lambda i: (i, 0)),
        ],
        out_shape=[
            jax.ShapeDtypeStruct((N, node_out), jnp.float32),
            jax.ShapeDtypeStruct((E // PACK, PACK * edge_out), jnp.float32),
        ],
        compiler_params=pltpu.CompilerParams(
            dimension_semantics=("parallel",),
        ),
    )(nodes, W1, b1.reshape(1, -1), W2, b2.reshape(1, -1), Wd_n, bd_n.reshape(1, -1),
      edges_p, Wde_bd, bde_t)

    out_edges = jnp.concatenate(
        [out_edges_p[:, k * edge_out:(k + 1) * edge_out] for k in range(PACK)], axis=0)
    new_globals = globals_ + DT
    return out_nodes, out_edges, new_globals


# speedup vs baseline: 1.8881x; 1.8881x over previous
import jax
import jax.numpy as jnp
from jax.experimental import pallas as pl
from jax.experimental.pallas import tpu as pltpu

N = 10000
E = 160000
DT = 1.0

GRID = 25
NODE_BLOCK = N // GRID
EDGE_BLOCK = E // GRID


def _fused_kernel(x_ref, w1_ref, b1_ref, w2_ref, b2_ref, wdn_ref, bdn_ref,
                  e_ref, wde_ref, bde_ref, on_ref, oe_ref):
    x = x_ref[...]
    h = jnp.dot(x, w1_ref[...], preferred_element_type=jnp.float32) + b1_ref[...]
    h = h * jax.nn.sigmoid(h)
    h = jnp.dot(h, w2_ref[...], preferred_element_type=jnp.float32) + b2_ref[...]
    h = h * jax.nn.sigmoid(h)
    on_ref[...] = jnp.dot(h, wdn_ref[...], preferred_element_type=jnp.float32) + bdn_ref[...]
    oe_ref[...] = jnp.dot(e_ref[...], wde_ref[...], preferred_element_type=jnp.float32) + bde_ref[...]


def kernel(nodes, edges, senders, receivers, globals_, W_enc_n, b_enc_n, W_enc_e, b_enc_e, W1, b1, W2, b2, Wd_n, bd_n, Wd_e, bd_e):
    d_feat = nodes.shape[1]
    latent = W1.shape[1]
    node_out = Wd_n.shape[1]
    d_edge = edges.shape[1]
    edge_out = Wd_e.shape[1]

    whole = lambda *shape: pl.BlockSpec(shape, lambda i: (0,) * len(shape))

    out_nodes, out_edges = pl.pallas_call(
        _fused_kernel,
        grid=(GRID,),
        in_specs=[
            pl.BlockSpec((NODE_BLOCK, d_feat), lambda i: (i, 0)),
            whole(d_feat, latent),
            whole(1, latent),
            whole(latent, latent),
            whole(1, latent),
            whole(latent, node_out),
            whole(1, node_out),
            pl.BlockSpec((EDGE_BLOCK, d_edge), lambda i: (i, 0)),
            whole(d_edge, edge_out),
            whole(1, edge_out),
        ],
        out_specs=[
            pl.BlockSpec((NODE_BLOCK, node_out), lambda i: (i, 0)),
            pl.BlockSpec((EDGE_BLOCK, edge_out), lambda i: (i, 0)),
        ],
        out_shape=[
            jax.ShapeDtypeStruct((N, node_out), jnp.float32),
            jax.ShapeDtypeStruct((E, edge_out), jnp.float32),
        ],
        compiler_params=pltpu.CompilerParams(
            dimension_semantics=("parallel",),
        ),
    )(nodes, W1, b1.reshape(1, -1), W2, b2.reshape(1, -1), Wd_n, bd_n.reshape(1, -1),
      edges, Wd_e, bd_e.reshape(1, -1))

    new_globals = globals_ + DT
    return out_nodes, out_edges, new_globals


# speedup vs baseline: 7.7342x; 4.0962x over previous
import jax
import jax.numpy as jnp
from jax.experimental import pallas as pl
from jax.experimental.pallas import tpu as pltpu

N = 10000
E = 160000
DT = 1.0

GRID = 25
NODE_BLOCK = N // GRID
EDGE_BLOCK = E // GRID


def _fused_kernel(x_ref, w1_ref, b1_ref, w2_ref, b2_ref, wdn_ref, bdn_ref,
                  et_ref, wdet_ref, bdet_ref, on_ref, oet_ref):
    x = x_ref[...]
    h = jnp.dot(x, w1_ref[...], preferred_element_type=jnp.float32) + b1_ref[...]
    h = h * jax.nn.sigmoid(h)
    h = jnp.dot(h, w2_ref[...], preferred_element_type=jnp.float32) + b2_ref[...]
    h = h * jax.nn.sigmoid(h)
    on_ref[...] = jnp.dot(h, wdn_ref[...], preferred_element_type=jnp.float32) + bdn_ref[...]
    oet_ref[...] = (jnp.dot(wdet_ref[...], et_ref[...], preferred_element_type=jnp.float32)
                    + bdet_ref[...])


def kernel(nodes, edges, senders, receivers, globals_, W_enc_n, b_enc_n, W_enc_e, b_enc_e, W1, b1, W2, b2, Wd_n, bd_n, Wd_e, bd_e):
    d_feat = nodes.shape[1]
    latent = W1.shape[1]
    node_out = Wd_n.shape[1]
    d_edge = edges.shape[1]
    edge_out = Wd_e.shape[1]

    edges_t = edges.T
    wde_t = Wd_e.T
    bde_c = bd_e.reshape(-1, 1)

    whole = lambda *shape: pl.BlockSpec(shape, lambda i: (0,) * len(shape))

    out_nodes, out_edges_t = pl.pallas_call(
        _fused_kernel,
        grid=(GRID,),
        in_specs=[
            pl.BlockSpec((NODE_BLOCK, d_feat), lambda i: (i, 0)),
            whole(d_feat, latent),
            whole(1, latent),
            whole(latent, latent),
            whole(1, latent),
            whole(latent, node_out),
            whole(1, node_out),
            pl.BlockSpec((d_edge, EDGE_BLOCK), lambda i: (0, i)),
            whole(edge_out, d_edge),
            whole(edge_out, 1),
        ],
        out_specs=[
            pl.BlockSpec((NODE_BLOCK, node_out), lambda i: (i, 0)),
            pl.BlockSpec((edge_out, EDGE_BLOCK), lambda i: (0, i)),
        ],
        out_shape=[
            jax.ShapeDtypeStruct((N, node_out), jnp.float32),
            jax.ShapeDtypeStruct((edge_out, E), jnp.float32),
        ],
        compiler_params=pltpu.CompilerParams(
            dimension_semantics=("parallel",),
        ),
    )(nodes, W1, b1.reshape(1, -1), W2, b2.reshape(1, -1), Wd_n, bd_n.reshape(1, -1),
      edges_t, wde_t, bde_c)

    out_edges = out_edges_t.T
    new_globals = globals_ + DT
    return out_nodes, out_edges, new_globals
